# Initial kernel scaffold; baseline (speedup 1.0000x reference)
import jax
import jax.numpy as jnp
from jax.experimental import pallas as pl


def kernel(x, edge_index, edge_weight, noisy_value, W1, b1, W2, b2, fc1_W, fc1_b, fc2_W, fc2_b, fc3_W, fc3_b):
    raise NotImplementedError("write your pallas kernel here")


# speedup vs baseline: 8.6742x; 8.6742x over previous
import jax
import jax.numpy as jnp
from jax.experimental import pallas as pl

N = 8
E = 32


def _bf(a):
    return a.astype(jnp.bfloat16)


def _dot(a, b):
    return jnp.dot(_bf(a), _bf(b), preferred_element_type=jnp.float32)


def _dot_exact(a, b):
    return jnp.dot(a, b, preferred_element_type=jnp.float32,
                   precision=jax.lax.Precision.HIGHEST)


def _fused_kernel(edge_ref, ew_ref, x_ref, noisy_ref, W1_ref, b1_ref,
                  W2_ref, b2_ref, fc1_ref, fc1b_bias_ref,
                  fc2_ref, fc2b_ref, fc3_ref, fc3b_ref, out_ref):
    src_row = edge_ref[0:1, :]
    dst_row = edge_ref[1:2, :]
    ew_row = ew_ref[...]

    iota_d = jax.lax.broadcasted_iota(jnp.int32, (N, E), 0)
    M = jnp.where(iota_d == dst_row, ew_row, 0.0)

    deg_col = jnp.sum(M, axis=1, keepdims=True) + 1.0
    dinv_col = jnp.where(deg_col > 0, 1.0 / jnp.sqrt(deg_col), 0.0)

    cols = []
    deg_row_parts = []
    for s in range(N):
        mask_s = (src_row == s)
        cols.append(jnp.sum(jnp.where(mask_s, M, 0.0), axis=1, keepdims=True))
        deg_row_parts.append(
            jnp.sum(jnp.where(dst_row == s, ew_row, 0.0), axis=1,
                    keepdims=True))
    A = jnp.concatenate(cols, axis=1)
    deg_row = jnp.concatenate(deg_row_parts, axis=1) + 1.0
    dinv_row = jnp.where(deg_row > 0, 1.0 / jnp.sqrt(deg_row), 0.0)

    eye = (jax.lax.broadcasted_iota(jnp.int32, (N, N), 0) ==
           jax.lax.broadcasted_iota(jnp.int32, (N, N), 1)).astype(jnp.float32)
    A_hat = dinv_col * (A + eye) * dinv_row

    Z = _dot(x_ref[...], W1_ref[...])
    H = jnp.maximum(_dot_exact(A_hat, Z) + b1_ref[...], 0.0)

    q = _dot(H, W2_ref[...])
    g = _dot_exact(A_hat, q) + b2_ref[...]

    gT = jnp.sum(jnp.where(eye > 0, g, 0.0), axis=0, keepdims=True)
    combined = jnp.concatenate([gT, noisy_ref[...]], axis=1)
    h1 = jnp.maximum(_dot(combined, fc1_ref[...]) + fc1b_bias_ref[...], 0.0)

    h2 = jnp.maximum(_dot(h1, fc2_ref[...]) + fc2b_ref[...], 0.0)

    out_ref[...] = (jnp.sum(h2 * fc3_ref[...], axis=1, keepdims=True)
                    + fc3b_ref[...])


def kernel(x, edge_index, edge_weight, noisy_value, W1, b1, W2, b2,
           fc1_W, fc1_b, fc2_W, fc2_b, fc3_W, fc3_b):
    ew_row = edge_weight.reshape(1, E)
    noisy = jax.lax.stop_gradient(noisy_value)
    args = (
        edge_index, ew_row, x, noisy,
        W1, b1.reshape(1, -1), W2, b2.reshape(1, -1),
        fc1_W, fc1_b.reshape(1, -1),
        fc2_W, fc2_b.reshape(1, -1), fc3_W.reshape(1, -1),
        fc3_b.reshape(1, -1),
    )
    return pl.pallas_call(
        _fused_kernel,
        out_shape=jax.ShapeDtypeStruct((1, 1), jnp.float32),
    )(*args)
